# baseline (speedup 1.0000x reference)
import functools

import numpy as np
import jax
import jax.numpy as jnp
from jax import lax
from jax.experimental import pallas as pl
from jax.experimental.pallas import tpu as pltpu
from jax.experimental.pallas import tpu_sc as plsc

_L = 16
_D = 2
_T = 2 ** 19
_B = 131072
_BASE = 16
_FINEST = 2048
_GROWTH = float(np.exp(np.log(_FINEST / _BASE) / (_L - 1)))
_RES = tuple(int(np.floor(_BASE * (_GROWTH ** l))) for l in range(_L))
_P1 = np.int32(2654435761 - 2 ** 32)
_P2 = np.int32(805459861)
_MASK = np.int32(_T - 1)

_NC = 2
_NS = 16
_NW = _NC * _NS
_PPW = _B // _NW
_CHUNK = 128
_NCH = _PPW // _CHUNK
_NSLOT = _L * 9
_GPC = _CHUNK // 16


def _encode_body(x_hbm, tbl_hbm, out_hbm, xbuf, idxbuf, idxrem, rows, outbuf,
                 gsem):
    cid = lax.axis_index("c")
    sid = lax.axis_index("s")
    wid = sid * _NC + cid

    def chunk_body(ch, carry):
        base = wid * _PPW + ch * _CHUNK
        pltpu.sync_copy(x_hbm.at[pl.ds(base * 3, _CHUNK * 3)], xbuf)

        def load_xyz(p):
            p3 = p * 3
            x0 = plsc.load_gather(xbuf, [p3])
            x1 = plsc.load_gather(xbuf, [p3 + 1])
            x2 = plsc.load_gather(xbuf, [p3 + 2])
            return x0, x1, x2

        def fire(l, b):
            pltpu.async_copy(tbl_hbm.at[idxbuf.at[l]], rows.at[b], gsem)

        def drain(l, b):
            pltpu.make_async_copy(
                tbl_hbm.at[idxbuf.at[l]], rows.at[b], gsem).wait()

        def build_idx(l):
            resf = jnp.float32(_RES[l])
            loff = jnp.int32(l * _T)

            def idx_one(goff, l=l, resf=resf, loff=loff):
                lanes = lax.iota(jnp.int32, 16)
                p = goff + lanes
                x0, x1, x2 = load_xyz(p)
                pos0 = x0 * resf
                pos1 = x1 * resf
                pos2 = x2 * resf
                i0 = pos0.astype(jnp.int32)
                i1 = pos1.astype(jnp.int32)
                i2 = pos2.astype(jnp.int32)
                a0 = i0
                a1 = i0 + 1
                b0 = i1 * _P1
                b1 = b0 + _P1
                c0 = i2 * _P2
                c1 = c0 + _P2
                for corner in range(8):
                    h = (a1 if corner & 1 else a0)
                    h = h ^ (b1 if corner & 2 else b0)
                    h = h ^ (c1 if corner & 4 else c0)
                    full = (h & _MASK) + loff
                    idxbuf[l, pl.ds(corner * _CHUNK + goff, 16)] = (
                        lax.shift_right_logical(full, 2))
                    idxrem[pl.ds((l * 9 + corner) * _CHUNK + goff, 16)] = (
                        (full & 3) * 2)
                t0 = pos0 + 0.5
                t1 = pos1 + 0.5
                t2 = pos2 + 0.5
                n0 = t0.astype(jnp.int32)
                n1 = t1.astype(jnp.int32)
                n2 = t2.astype(jnp.int32)
                n0 = n0 - jnp.where(n0.astype(jnp.float32) == t0, n0 & 1, 0)
                n1 = n1 - jnp.where(n1.astype(jnp.float32) == t1, n1 & 1, 0)
                n2 = n2 - jnp.where(n2.astype(jnp.float32) == t2, n2 & 1, 0)
                hn = n0 ^ (n1 * _P1) ^ (n2 * _P2)
                fulln = (hn & _MASK) + loff
                idxbuf[l, pl.ds(8 * _CHUNK + goff, 16)] = (
                    lax.shift_right_logical(fulln, 2))
                idxrem[pl.ds((l * 9 + 8) * _CHUNK + goff, 16)] = (
                    (fulln & 3) * 2)

            def idx_body(g, c):
                idx_one(g * 32)
                idx_one(g * 32 + 16)
                return c

            lax.fori_loop(0, _GPC // 2, idx_body, 0)

        def interp(l, b):
            resf = jnp.float32(_RES[l])

            def interp_one(goff, l=l, resf=resf, b=b):
                lanes = lax.iota(jnp.int32, 16)
                p = goff + lanes
                zero = jnp.zeros((16,), jnp.int32)
                one = zero + 1
                x0, x1, x2 = load_xyz(p)
                pos0 = x0 * resf
                pos1 = x1 * resf
                pos2 = x2 * resf
                w0 = pos0 - pos0.astype(jnp.int32).astype(jnp.float32)
                w1 = pos1 - pos1.astype(jnp.int32).astype(jnp.float32)
                w2 = pos2 - pos2.astype(jnp.int32).astype(jnp.float32)
                u0 = 1.0 - w0
                u1 = 1.0 - w1
                u2 = 1.0 - w2
                m00 = u0 * u1
                m10 = w0 * u1
                m01 = u0 * w1
                m11 = w0 * w1
                mxy = (m00, m10, m01, m11)
                feat0 = jnp.zeros((16,), jnp.float32)
                feat1 = jnp.zeros((16,), jnp.float32)
                for corner in range(8):
                    srow = corner * _CHUNK + p
                    wc = mxy[corner & 3] * (w2 if corner & 4 else u2)
                    r2 = plsc.load_gather(
                        idxrem, [(l * 9 + corner) * _CHUNK + p])
                    f0 = plsc.load_gather(rows.at[b], [srow, r2])
                    f1 = plsc.load_gather(rows.at[b], [srow, r2 + 1])
                    feat0 = feat0 + f0 * wc
                    feat1 = feat1 + f1 * wc
                srow = 8 * _CHUNK + p
                r2 = plsc.load_gather(idxrem, [(l * 9 + 8) * _CHUNK + p])
                st0 = plsc.load_gather(rows.at[b], [srow, r2])
                st1 = plsc.load_gather(rows.at[b], [srow, r2 + 1])
                p32 = p * (2 * _L)
                plsc.store_scatter(outbuf, [p32 + (2 * l)], feat0 - st0)
                plsc.store_scatter(outbuf, [p32 + (2 * l + 1)], feat1 - st1)

            def interp_body(g, c):
                interp_one(g * 32)
                interp_one(g * 32 + 16)
                return c

            lax.fori_loop(0, _GPC // 2, interp_body, 0)

        _NBUF = 4
        for l in range(_L):
            build_idx(l)
            fire(l, l % _NBUF)
            if l >= _NBUF - 1:
                lw = l - (_NBUF - 1)
                drain(lw, lw % _NBUF)
                interp(lw, lw % _NBUF)
        for lw in range(_L - (_NBUF - 1), _L):
            drain(lw, lw % _NBUF)
            interp(lw, lw % _NBUF)

        pltpu.sync_copy(outbuf, out_hbm.at[pl.ds(base * 2 * _L, _CHUNK * 2 * _L)])
        return carry

    lax.fori_loop(0, _NCH, chunk_body, 0)


_encode = functools.partial(
    pl.kernel,
    out_type=jax.ShapeDtypeStruct((_B * 2 * _L,), jnp.float32),
    mesh=plsc.VectorSubcoreMesh(core_axis_name="c", subcore_axis_name="s"),
    compiler_params=pltpu.CompilerParams(
        needs_layout_passes=False, use_tc_tiling_on_sc=False),
    scratch_types=[
        pltpu.VMEM((_CHUNK * 3,), jnp.float32),
        pltpu.VMEM((_L, 9 * _CHUNK), jnp.int32),
        pltpu.VMEM((_NSLOT * _CHUNK,), jnp.int32),
        pltpu.VMEM((4, 9 * _CHUNK, 8), jnp.float32),
        pltpu.VMEM((_CHUNK * 2 * _L,), jnp.float32),
        pltpu.SemaphoreType.DMA,
    ],
)(_encode_body)


@jax.jit
def kernel(in_tensor, table):
    tbl = table.reshape(_L * _T // 4, 8) + 0.0
    flat = _encode(in_tensor.reshape(_B * 3) + 0.0, tbl)
    return flat.reshape(_B, 2 * _L)


# speedup vs baseline: 11.1497x; 11.1497x over previous
import functools

import numpy as np
import jax
import jax.numpy as jnp
from jax import lax
from jax.experimental import pallas as pl
from jax.experimental.pallas import tpu as pltpu
from jax.experimental.pallas import tpu_sc as plsc

_L = 16
_D = 2
_T = 2 ** 19
_B = 131072
_BASE = 16
_FINEST = 2048
_GROWTH = float(np.exp(np.log(_FINEST / _BASE) / (_L - 1)))
_RES = tuple(int(np.floor(_BASE * (_GROWTH ** l))) for l in range(_L))
_P1 = np.int32(2654435761 - 2 ** 32)
_P2 = np.int32(805459861)
_MASK = np.int32(_T - 1)

_NC = 2
_NS = 16
_NW = _NC * _NS
_PPW = _B // _NW
_CHUNK = 128
_NCH = _PPW // _CHUNK
_NSLOT = _L * 9
_GPC = _CHUNK // 16


def _encode_body(x_hbm, tbl_hbm, out_hbm, xbuf, idxbuf, idxrem, rows, outbuf,
                 gsem):
    cid = lax.axis_index("c")
    sid = lax.axis_index("s")
    wid = sid * _NC + cid

    def chunk_body(ch, carry):
        base = wid * _PPW + ch * _CHUNK
        pltpu.sync_copy(x_hbm.at[pl.ds(base * 3, _CHUNK * 3)], xbuf)

        def load_xyz(p):
            p3 = p * 3
            x0 = plsc.load_gather(xbuf, [p3])
            x1 = plsc.load_gather(xbuf, [p3 + 1])
            x2 = plsc.load_gather(xbuf, [p3 + 2])
            return x0, x1, x2

        def fire(l, b):
            pltpu.async_copy(tbl_hbm.at[idxbuf.at[l]], rows.at[b], gsem)

        def drain(l, b):
            pltpu.make_async_copy(
                tbl_hbm.at[idxbuf.at[l]], rows.at[b], gsem).wait()

        def build_idx(l):
            resf = jnp.float32(_RES[l])
            loff = jnp.int32(l * _T)

            def idx_one(goff, l=l, resf=resf, loff=loff):
                lanes = lax.iota(jnp.int32, 16)
                p = goff + lanes
                x0, x1, x2 = load_xyz(p)
                pos0 = x0 * resf
                pos1 = x1 * resf
                pos2 = x2 * resf
                i0 = pos0.astype(jnp.int32)
                i1 = pos1.astype(jnp.int32)
                i2 = pos2.astype(jnp.int32)
                a0 = i0
                a1 = i0 + 1
                b0 = i1 * _P1
                b1 = b0 + _P1
                c0 = i2 * _P2
                c1 = c0 + _P2
                for corner in range(8):
                    h = (a1 if corner & 1 else a0)
                    h = h ^ (b1 if corner & 2 else b0)
                    h = h ^ (c1 if corner & 4 else c0)
                    full = (h & _MASK) + loff
                    idxbuf[l, pl.ds(corner * _CHUNK + goff, 16)] = (
                        lax.shift_right_logical(full, 2))
                    idxrem[pl.ds((l * 9 + corner) * _CHUNK + goff, 16)] = (
                        (full & 3) * 2)
                t0 = pos0 + 0.5
                t1 = pos1 + 0.5
                t2 = pos2 + 0.5
                n0 = t0.astype(jnp.int32)
                n1 = t1.astype(jnp.int32)
                n2 = t2.astype(jnp.int32)
                n0 = n0 - jnp.where(n0.astype(jnp.float32) == t0, n0 & 1, 0)
                n1 = n1 - jnp.where(n1.astype(jnp.float32) == t1, n1 & 1, 0)
                n2 = n2 - jnp.where(n2.astype(jnp.float32) == t2, n2 & 1, 0)
                hn = n0 ^ (n1 * _P1) ^ (n2 * _P2)
                fulln = (hn & _MASK) + loff
                idxbuf[l, pl.ds(8 * _CHUNK + goff, 16)] = (
                    lax.shift_right_logical(fulln, 2))
                idxrem[pl.ds((l * 9 + 8) * _CHUNK + goff, 16)] = (
                    (fulln & 3) * 2)

            def idx_body(g, c):
                idx_one(g * 32)
                idx_one(g * 32 + 16)
                return c

            lax.fori_loop(0, _GPC // 2, idx_body, 0)

        def interp(l, b):
            resf = jnp.float32(_RES[l])

            def interp_one(goff, l=l, resf=resf, b=b):
                lanes = lax.iota(jnp.int32, 16)
                p = goff + lanes
                zero = jnp.zeros((16,), jnp.int32)
                one = zero + 1
                x0, x1, x2 = load_xyz(p)
                pos0 = x0 * resf
                pos1 = x1 * resf
                pos2 = x2 * resf
                w0 = pos0 - pos0.astype(jnp.int32).astype(jnp.float32)
                w1 = pos1 - pos1.astype(jnp.int32).astype(jnp.float32)
                w2 = pos2 - pos2.astype(jnp.int32).astype(jnp.float32)
                u0 = 1.0 - w0
                u1 = 1.0 - w1
                u2 = 1.0 - w2
                m00 = u0 * u1
                m10 = w0 * u1
                m01 = u0 * w1
                m11 = w0 * w1
                mxy = (m00, m10, m01, m11)
                feat0 = jnp.zeros((16,), jnp.float32)
                feat1 = jnp.zeros((16,), jnp.float32)
                for corner in range(8):
                    srow = corner * _CHUNK + p
                    wc = mxy[corner & 3] * (w2 if corner & 4 else u2)
                    r2 = plsc.load_gather(
                        idxrem, [(l * 9 + corner) * _CHUNK + p])
                    f0 = plsc.load_gather(rows.at[b], [srow, r2])
                    f1 = plsc.load_gather(rows.at[b], [srow, r2 + 1])
                    feat0 = feat0 + f0 * wc
                    feat1 = feat1 + f1 * wc
                srow = 8 * _CHUNK + p
                r2 = plsc.load_gather(idxrem, [(l * 9 + 8) * _CHUNK + p])
                st0 = plsc.load_gather(rows.at[b], [srow, r2])
                st1 = plsc.load_gather(rows.at[b], [srow, r2 + 1])
                p32 = p * (2 * _L)
                plsc.store_scatter(outbuf, [p32 + (2 * l)], feat0 - st0)
                plsc.store_scatter(outbuf, [p32 + (2 * l + 1)], feat1 - st1)

            def interp_body(g, c):
                interp_one(g * 32)
                interp_one(g * 32 + 16)
                return c

            lax.fori_loop(0, _GPC // 2, interp_body, 0)

        _NBUF = 4
        for l in range(_L):
            build_idx(l)
            fire(l, l % _NBUF)
            if l >= _NBUF - 1:
                lw = l - (_NBUF - 1)
                drain(lw, lw % _NBUF)
                interp(lw, lw % _NBUF)
        for lw in range(_L - (_NBUF - 1), _L):
            drain(lw, lw % _NBUF)
            interp(lw, lw % _NBUF)

        pltpu.sync_copy(outbuf, out_hbm.at[pl.ds(base * 2 * _L, _CHUNK * 2 * _L)])
        return carry

    lax.fori_loop(0, _NCH, chunk_body, 0)


def _interleave_body(src_hbm, out_hbm, inbuf, outbuf):
    cid = lax.axis_index("c")
    sid = lax.axis_index("s")
    wid = sid * _NC + cid
    rows_per_tile = (_L * _T * _D // 128) // _NW
    pairs_per_tile = rows_per_tile // 2
    _PBLK = 32

    def blk_body(blk, carry):
        pbase = wid * pairs_per_tile + blk * _PBLK
        pltpu.sync_copy(src_hbm.at[pl.ds(pbase * 2, 2 * _PBLK), :], inbuf)

        def pair_body(pp, c):
            for v in range(8):
                lanes = lax.iota(jnp.int32, 16)
                bv = pp * 256 + v * 32 + lanes * 2
                d0 = inbuf[2 * pp, pl.ds(v * 16, 16)]
                d1 = inbuf[2 * pp + 1, pl.ds(v * 16, 16)]
                plsc.store_scatter(outbuf, [bv], d0)
                plsc.store_scatter(outbuf, [bv + 1], d1)
            return c

        lax.fori_loop(0, _PBLK, pair_body, 0)
        pltpu.sync_copy(outbuf, out_hbm.at[pl.ds(pbase * 256, _PBLK * 256)])
        return carry

    lax.fori_loop(0, pairs_per_tile // _PBLK, blk_body, 0)


_interleave = functools.partial(
    pl.kernel,
    out_type=jax.ShapeDtypeStruct((_L * _T * _D,), jnp.float32),
    mesh=plsc.VectorSubcoreMesh(core_axis_name="c", subcore_axis_name="s"),
    compiler_params=pltpu.CompilerParams(
        needs_layout_passes=False, use_tc_tiling_on_sc=False),
    scratch_types=[
        pltpu.VMEM((64, 128), jnp.float32),
        pltpu.VMEM((32 * 256,), jnp.float32),
    ],
)(_interleave_body)


_encode = functools.partial(
    pl.kernel,
    out_type=jax.ShapeDtypeStruct((_B * 2 * _L,), jnp.float32),
    mesh=plsc.VectorSubcoreMesh(core_axis_name="c", subcore_axis_name="s"),
    compiler_params=pltpu.CompilerParams(
        needs_layout_passes=False, use_tc_tiling_on_sc=False),
    scratch_types=[
        pltpu.VMEM((_CHUNK * 3,), jnp.float32),
        pltpu.VMEM((_L, 9 * _CHUNK), jnp.int32),
        pltpu.VMEM((_NSLOT * _CHUNK,), jnp.int32),
        pltpu.VMEM((4, 9 * _CHUNK, 8), jnp.float32),
        pltpu.VMEM((_CHUNK * 2 * _L,), jnp.float32),
        pltpu.SemaphoreType.DMA,
    ],
)(_encode_body)


@jax.jit
def kernel(in_tensor, table):
    tblv = (table.reshape(_L, _T // 128, 128, _D)
            .transpose(0, 1, 3, 2)
            .reshape(_L * _T * _D // 128, 128))
    tbl8 = _interleave(tblv).reshape(_L * _T // 4, 8)
    flat = _encode(in_tensor.reshape(_B * 3), tbl8)
    return flat.reshape(_B, 2 * _L)


# speedup vs baseline: 11.8799x; 1.0655x over previous
import functools

import numpy as np
import jax
import jax.numpy as jnp
from jax import lax
from jax.experimental import pallas as pl
from jax.experimental.pallas import tpu as pltpu
from jax.experimental.pallas import tpu_sc as plsc

_L = 16
_D = 2
_T = 2 ** 19
_B = 131072
_BASE = 16
_FINEST = 2048
_GROWTH = float(np.exp(np.log(_FINEST / _BASE) / (_L - 1)))
_RES = tuple(int(np.floor(_BASE * (_GROWTH ** l))) for l in range(_L))
_P1 = np.int32(2654435761 - 2 ** 32)
_P2 = np.int32(805459861)
_MASK = np.int32(_T - 1)

_NC = 2
_NS = 16
_NW = _NC * _NS
_PPW = _B // _NW
_CHUNK = 128
_NCH = _PPW // _CHUNK
_NSLOT = _L * 9
_GPC = _CHUNK // 16


def _encode_body(x_hbm, tbl_hbm, out_hbm, xbuf, idxbuf, idxrem, rows, outbuf,
                 gsem):
    cid = lax.axis_index("c")
    sid = lax.axis_index("s")
    wid = sid * _NC + cid

    def chunk_body(ch, carry):
        base = wid * _PPW + ch * _CHUNK
        pltpu.sync_copy(x_hbm.at[pl.ds(base * 3, _CHUNK * 3)], xbuf)

        def load_xyz(p):
            p3 = p * 3
            x0 = plsc.load_gather(xbuf, [p3])
            x1 = plsc.load_gather(xbuf, [p3 + 1])
            x2 = plsc.load_gather(xbuf, [p3 + 2])
            return x0, x1, x2

        def fire(l, b):
            pltpu.async_copy(tbl_hbm.at[idxbuf.at[l]], rows.at[b], gsem)

        def drain(l, b):
            pltpu.make_async_copy(
                tbl_hbm.at[idxbuf.at[l]], rows.at[b], gsem).wait()

        def build_idx(l):
            resf = jnp.float32(_RES[l])
            loff = jnp.int32(l * _T)

            def idx_one(goff, l=l, resf=resf, loff=loff):
                lanes = lax.iota(jnp.int32, 16)
                p = goff + lanes
                x0, x1, x2 = load_xyz(p)
                pos0 = x0 * resf
                pos1 = x1 * resf
                pos2 = x2 * resf
                i0 = pos0.astype(jnp.int32)
                i1 = pos1.astype(jnp.int32)
                i2 = pos2.astype(jnp.int32)
                a0 = i0
                a1 = i0 + 1
                b0 = i1 * _P1
                b1 = b0 + _P1
                c0 = i2 * _P2
                c1 = c0 + _P2
                for corner in range(8):
                    h = (a1 if corner & 1 else a0)
                    h = h ^ (b1 if corner & 2 else b0)
                    h = h ^ (c1 if corner & 4 else c0)
                    full = (h & _MASK) + loff
                    idxbuf[l, pl.ds(corner * _CHUNK + goff, 16)] = (
                        lax.shift_right_logical(full, 2))
                    idxrem[pl.ds((l * 9 + corner) * _CHUNK + goff, 16)] = (
                        (full & 3) * 2)
                t0 = pos0 + 0.5
                t1 = pos1 + 0.5
                t2 = pos2 + 0.5
                n0 = t0.astype(jnp.int32)
                n1 = t1.astype(jnp.int32)
                n2 = t2.astype(jnp.int32)
                n0 = n0 - jnp.where(n0.astype(jnp.float32) == t0, n0 & 1, 0)
                n1 = n1 - jnp.where(n1.astype(jnp.float32) == t1, n1 & 1, 0)
                n2 = n2 - jnp.where(n2.astype(jnp.float32) == t2, n2 & 1, 0)
                hn = n0 ^ (n1 * _P1) ^ (n2 * _P2)
                fulln = (hn & _MASK) + loff
                idxbuf[l, pl.ds(8 * _CHUNK + goff, 16)] = (
                    lax.shift_right_logical(fulln, 2))
                idxrem[pl.ds((l * 9 + 8) * _CHUNK + goff, 16)] = (
                    (fulln & 3) * 2)

            def idx_body(g, c):
                idx_one(g * 32)
                idx_one(g * 32 + 16)
                return c

            lax.fori_loop(0, _GPC // 2, idx_body, 0)

        def interp(l, b):
            resf = jnp.float32(_RES[l])

            def interp_one(goff, l=l, resf=resf, b=b):
                lanes = lax.iota(jnp.int32, 16)
                p = goff + lanes
                zero = jnp.zeros((16,), jnp.int32)
                one = zero + 1
                x0, x1, x2 = load_xyz(p)
                pos0 = x0 * resf
                pos1 = x1 * resf
                pos2 = x2 * resf
                w0 = pos0 - pos0.astype(jnp.int32).astype(jnp.float32)
                w1 = pos1 - pos1.astype(jnp.int32).astype(jnp.float32)
                w2 = pos2 - pos2.astype(jnp.int32).astype(jnp.float32)
                u0 = 1.0 - w0
                u1 = 1.0 - w1
                u2 = 1.0 - w2
                m00 = u0 * u1
                m10 = w0 * u1
                m01 = u0 * w1
                m11 = w0 * w1
                mxy = (m00, m10, m01, m11)
                feat0 = jnp.zeros((16,), jnp.float32)
                feat1 = jnp.zeros((16,), jnp.float32)
                for corner in range(8):
                    srow = corner * _CHUNK + p
                    wc = mxy[corner & 3] * (w2 if corner & 4 else u2)
                    r2 = plsc.load_gather(
                        idxrem, [(l * 9 + corner) * _CHUNK + p])
                    f0 = plsc.load_gather(rows.at[b], [srow, r2])
                    f1 = plsc.load_gather(rows.at[b], [srow, r2 + 1])
                    feat0 = feat0 + f0 * wc
                    feat1 = feat1 + f1 * wc
                srow = 8 * _CHUNK + p
                r2 = plsc.load_gather(idxrem, [(l * 9 + 8) * _CHUNK + p])
                st0 = plsc.load_gather(rows.at[b], [srow, r2])
                st1 = plsc.load_gather(rows.at[b], [srow, r2 + 1])
                p32 = p * (2 * _L)
                plsc.store_scatter(outbuf, [p32 + (2 * l)], feat0 - st0)
                plsc.store_scatter(outbuf, [p32 + (2 * l + 1)], feat1 - st1)

            def interp_body(g, c):
                interp_one(g * 32)
                interp_one(g * 32 + 16)
                return c

            lax.fori_loop(0, _GPC // 2, interp_body, 0)

        _NBUF = 4
        for l in range(_L):
            build_idx(l)
            fire(l, l % _NBUF)
            if l >= _NBUF - 1:
                lw = l - (_NBUF - 1)
                drain(lw, lw % _NBUF)
                interp(lw, lw % _NBUF)
        for lw in range(_L - (_NBUF - 1), _L):
            drain(lw, lw % _NBUF)
            interp(lw, lw % _NBUF)

        pltpu.sync_copy(outbuf, out_hbm.at[pl.ds(base * 2 * _L, _CHUNK * 2 * _L)])
        return carry

    lax.fori_loop(0, _NCH, chunk_body, 0)


def _interleave_body(src_hbm, out_hbm, inbuf, outbuf, insem, outsem):
    cid = lax.axis_index("c")
    sid = lax.axis_index("s")
    wid = sid * _NC + cid
    rows_per_tile = (_L * _T * _D // 128) // _NW
    pairs_per_tile = rows_per_tile // 2
    _PBLK = 32
    nblk = pairs_per_tile // _PBLK

    def in_copy(blk, b):
        pbase = wid * pairs_per_tile + blk * _PBLK
        return pltpu.make_async_copy(
            src_hbm.at[pl.ds(pbase * 2, 2 * _PBLK), :], inbuf.at[b], insem)

    def out_copy(blk, b):
        pbase = wid * pairs_per_tile + blk * _PBLK
        return pltpu.make_async_copy(
            outbuf.at[b], out_hbm.at[pl.ds(pbase * 256, _PBLK * 256)], outsem)

    in_copy(0, 0).start()
    for blk in range(nblk):
        b = blk % 2
        in_copy(blk, b).wait()
        if blk + 1 < nblk:
            in_copy(blk + 1, (blk + 1) % 2).start()
        if blk >= 2:
            out_copy(blk - 2, b).wait()

        def pair_body(pp, c, b=b):
            for v in range(8):
                lanes = lax.iota(jnp.int32, 16)
                bv = pp * 256 + v * 32 + lanes * 2
                d0 = inbuf[b, 2 * pp, pl.ds(v * 16, 16)]
                d1 = inbuf[b, 2 * pp + 1, pl.ds(v * 16, 16)]
                plsc.store_scatter(outbuf.at[b], [bv], d0)
                plsc.store_scatter(outbuf.at[b], [bv + 1], d1)
            return c

        lax.fori_loop(0, _PBLK, pair_body, 0)
        out_copy(blk, b).start()
    out_copy(nblk - 2, (nblk - 2) % 2).wait()
    out_copy(nblk - 1, (nblk - 1) % 2).wait()


_interleave = functools.partial(
    pl.kernel,
    out_type=jax.ShapeDtypeStruct((_L * _T * _D,), jnp.float32),
    mesh=plsc.VectorSubcoreMesh(core_axis_name="c", subcore_axis_name="s"),
    compiler_params=pltpu.CompilerParams(
        needs_layout_passes=False, use_tc_tiling_on_sc=False),
    scratch_types=[
        pltpu.VMEM((2, 64, 128), jnp.float32),
        pltpu.VMEM((2, 32 * 256), jnp.float32),
        pltpu.SemaphoreType.DMA,
        pltpu.SemaphoreType.DMA,
    ],
)(_interleave_body)


_encode = functools.partial(
    pl.kernel,
    out_type=jax.ShapeDtypeStruct((_B * 2 * _L,), jnp.float32),
    mesh=plsc.VectorSubcoreMesh(core_axis_name="c", subcore_axis_name="s"),
    compiler_params=pltpu.CompilerParams(
        needs_layout_passes=False, use_tc_tiling_on_sc=False),
    scratch_types=[
        pltpu.VMEM((_CHUNK * 3,), jnp.float32),
        pltpu.VMEM((_L, 9 * _CHUNK), jnp.int32),
        pltpu.VMEM((_NSLOT * _CHUNK,), jnp.int32),
        pltpu.VMEM((4, 9 * _CHUNK, 8), jnp.float32),
        pltpu.VMEM((_CHUNK * 2 * _L,), jnp.float32),
        pltpu.SemaphoreType.DMA,
    ],
)(_encode_body)


@jax.jit
def kernel(in_tensor, table):
    tblv = (table.reshape(_L, _T // 128, 128, _D)
            .transpose(0, 1, 3, 2)
            .reshape(_L * _T * _D // 128, 128))
    tbl8 = _interleave(tblv).reshape(_L * _T // 4, 8)
    flat = _encode(in_tensor.reshape(_B * 3), tbl8)
    return flat.reshape(_B, 2 * _L)


# speedup vs baseline: 12.1317x; 1.0212x over previous
import functools

import numpy as np
import jax
import jax.numpy as jnp
from jax import lax
from jax.experimental import pallas as pl
from jax.experimental.pallas import tpu as pltpu
from jax.experimental.pallas import tpu_sc as plsc

_L = 16
_D = 2
_T = 2 ** 19
_B = 131072
_BASE = 16
_FINEST = 2048
_GROWTH = float(np.exp(np.log(_FINEST / _BASE) / (_L - 1)))
_RES = tuple(int(np.floor(_BASE * (_GROWTH ** l))) for l in range(_L))
_P1 = np.int32(2654435761 - 2 ** 32)
_P2 = np.int32(805459861)
_MASK = np.int32(_T - 1)

_NC = 2
_NS = 16
_NW = _NC * _NS
_PPW = _B // _NW
_CHUNK = 128
_NCH = _PPW // _CHUNK
_NSLOT = _L * 9
_GPC = _CHUNK // 16


def _encode_body(x_hbm, tbl_hbm, out_hbm, xbuf, idxbuf, idxrem, rows, outbuf,
                 gsem):
    cid = lax.axis_index("c")
    sid = lax.axis_index("s")
    wid = sid * _NC + cid

    def chunk_body(ch, carry):
        base = wid * _PPW + ch * _CHUNK
        pltpu.sync_copy(x_hbm.at[pl.ds(base * 3, _CHUNK * 3)], xbuf)

        def load_xyz(p):
            p3 = p * 3
            x0 = plsc.load_gather(xbuf, [p3])
            x1 = plsc.load_gather(xbuf, [p3 + 1])
            x2 = plsc.load_gather(xbuf, [p3 + 2])
            return x0, x1, x2

        def fire(l, b):
            pltpu.async_copy(tbl_hbm.at[idxbuf.at[l]], rows.at[b], gsem)

        def drain(l, b):
            pltpu.make_async_copy(
                tbl_hbm.at[idxbuf.at[l]], rows.at[b], gsem).wait()

        def build_idx(l):
            resf = jnp.float32(_RES[l])
            loff = jnp.int32(l * _T)

            def idx_one(goff, l=l, resf=resf, loff=loff):
                lanes = lax.iota(jnp.int32, 16)
                p = goff + lanes
                x0, x1, x2 = load_xyz(p)
                pos0 = x0 * resf
                pos1 = x1 * resf
                pos2 = x2 * resf
                i0 = pos0.astype(jnp.int32)
                i1 = pos1.astype(jnp.int32)
                i2 = pos2.astype(jnp.int32)
                a0 = i0
                a1 = i0 + 1
                b0 = i1 * _P1
                b1 = b0 + _P1
                c0 = i2 * _P2
                c1 = c0 + _P2
                for corner in range(8):
                    h = (a1 if corner & 1 else a0)
                    h = h ^ (b1 if corner & 2 else b0)
                    h = h ^ (c1 if corner & 4 else c0)
                    full = (h & _MASK) + loff
                    idxbuf[l, pl.ds(corner * _CHUNK + goff, 16)] = (
                        lax.shift_right_logical(full, 3))
                    idxrem[pl.ds((l * 9 + corner) * _CHUNK + goff, 16)] = (
                        (full & 7) * 2)
                t0 = pos0 + 0.5
                t1 = pos1 + 0.5
                t2 = pos2 + 0.5
                n0 = t0.astype(jnp.int32)
                n1 = t1.astype(jnp.int32)
                n2 = t2.astype(jnp.int32)
                n0 = n0 - jnp.where(n0.astype(jnp.float32) == t0, n0 & 1, 0)
                n1 = n1 - jnp.where(n1.astype(jnp.float32) == t1, n1 & 1, 0)
                n2 = n2 - jnp.where(n2.astype(jnp.float32) == t2, n2 & 1, 0)
                hn = n0 ^ (n1 * _P1) ^ (n2 * _P2)
                fulln = (hn & _MASK) + loff
                idxbuf[l, pl.ds(8 * _CHUNK + goff, 16)] = (
                    lax.shift_right_logical(fulln, 3))
                idxrem[pl.ds((l * 9 + 8) * _CHUNK + goff, 16)] = (
                    (fulln & 7) * 2)

            def idx_body(g, c):
                idx_one(g * 32)
                idx_one(g * 32 + 16)
                return c

            lax.fori_loop(0, _GPC // 2, idx_body, 0)

        def interp(l, b):
            resf = jnp.float32(_RES[l])

            def interp_one(goff, l=l, resf=resf, b=b):
                lanes = lax.iota(jnp.int32, 16)
                p = goff + lanes
                zero = jnp.zeros((16,), jnp.int32)
                one = zero + 1
                x0, x1, x2 = load_xyz(p)
                pos0 = x0 * resf
                pos1 = x1 * resf
                pos2 = x2 * resf
                w0 = pos0 - pos0.astype(jnp.int32).astype(jnp.float32)
                w1 = pos1 - pos1.astype(jnp.int32).astype(jnp.float32)
                w2 = pos2 - pos2.astype(jnp.int32).astype(jnp.float32)
                u0 = 1.0 - w0
                u1 = 1.0 - w1
                u2 = 1.0 - w2
                m00 = u0 * u1
                m10 = w0 * u1
                m01 = u0 * w1
                m11 = w0 * w1
                mxy = (m00, m10, m01, m11)
                feat0 = jnp.zeros((16,), jnp.float32)
                feat1 = jnp.zeros((16,), jnp.float32)
                for corner in range(8):
                    srow = corner * _CHUNK + p
                    wc = mxy[corner & 3] * (w2 if corner & 4 else u2)
                    r2 = plsc.load_gather(
                        idxrem, [(l * 9 + corner) * _CHUNK + p])
                    f0 = plsc.load_gather(rows.at[b], [srow, r2])
                    f1 = plsc.load_gather(rows.at[b], [srow, r2 + 1])
                    feat0 = feat0 + f0 * wc
                    feat1 = feat1 + f1 * wc
                srow = 8 * _CHUNK + p
                r2 = plsc.load_gather(idxrem, [(l * 9 + 8) * _CHUNK + p])
                st0 = plsc.load_gather(rows.at[b], [srow, r2])
                st1 = plsc.load_gather(rows.at[b], [srow, r2 + 1])
                p32 = p * (2 * _L)
                plsc.store_scatter(outbuf, [p32 + (2 * l)], feat0 - st0)
                plsc.store_scatter(outbuf, [p32 + (2 * l + 1)], feat1 - st1)

            def interp_body(g, c):
                interp_one(g * 32)
                interp_one(g * 32 + 16)
                return c

            lax.fori_loop(0, _GPC // 2, interp_body, 0)

        _NBUF = 4
        for l in range(_L):
            build_idx(l)
            fire(l, l % _NBUF)
            if l >= _NBUF - 1:
                lw = l - (_NBUF - 1)
                drain(lw, lw % _NBUF)
                interp(lw, lw % _NBUF)
        for lw in range(_L - (_NBUF - 1), _L):
            drain(lw, lw % _NBUF)
            interp(lw, lw % _NBUF)

        pltpu.sync_copy(outbuf, out_hbm.at[pl.ds(base * 2 * _L, _CHUNK * 2 * _L)])
        return carry

    lax.fori_loop(0, _NCH, chunk_body, 0)


def _interleave_body(src_hbm, out_hbm, inbuf, outbuf, insem, outsem):
    cid = lax.axis_index("c")
    sid = lax.axis_index("s")
    wid = sid * _NC + cid
    rows_per_tile = (_L * _T * _D // 128) // _NW
    pairs_per_tile = rows_per_tile // 2
    _PBLK = 32
    nblk = pairs_per_tile // _PBLK

    def in_copy(blk, b):
        pbase = wid * pairs_per_tile + blk * _PBLK
        return pltpu.make_async_copy(
            src_hbm.at[pl.ds(pbase * 2, 2 * _PBLK), :], inbuf.at[b], insem)

    def out_copy(blk, b):
        pbase = wid * pairs_per_tile + blk * _PBLK
        return pltpu.make_async_copy(
            outbuf.at[b], out_hbm.at[pl.ds(pbase * 256, _PBLK * 256)], outsem)

    in_copy(0, 0).start()
    for blk in range(nblk):
        b = blk % 2
        in_copy(blk, b).wait()
        if blk + 1 < nblk:
            in_copy(blk + 1, (blk + 1) % 2).start()
        if blk >= 2:
            out_copy(blk - 2, b).wait()

        def pair_body(pp, c, b=b):
            for v in range(8):
                lanes = lax.iota(jnp.int32, 16)
                bv = pp * 256 + v * 32 + lanes * 2
                d0 = inbuf[b, 2 * pp, pl.ds(v * 16, 16)]
                d1 = inbuf[b, 2 * pp + 1, pl.ds(v * 16, 16)]
                plsc.store_scatter(outbuf.at[b], [bv], d0)
                plsc.store_scatter(outbuf.at[b], [bv + 1], d1)
            return c

        lax.fori_loop(0, _PBLK, pair_body, 0)
        out_copy(blk, b).start()
    out_copy(nblk - 2, (nblk - 2) % 2).wait()
    out_copy(nblk - 1, (nblk - 1) % 2).wait()


_interleave = functools.partial(
    pl.kernel,
    out_type=jax.ShapeDtypeStruct((_L * _T * _D,), jnp.float32),
    mesh=plsc.VectorSubcoreMesh(core_axis_name="c", subcore_axis_name="s"),
    compiler_params=pltpu.CompilerParams(
        needs_layout_passes=False, use_tc_tiling_on_sc=False),
    scratch_types=[
        pltpu.VMEM((2, 64, 128), jnp.float32),
        pltpu.VMEM((2, 32 * 256), jnp.float32),
        pltpu.SemaphoreType.DMA,
        pltpu.SemaphoreType.DMA,
    ],
)(_interleave_body)


_encode = functools.partial(
    pl.kernel,
    out_type=jax.ShapeDtypeStruct((_B * 2 * _L,), jnp.float32),
    mesh=plsc.VectorSubcoreMesh(core_axis_name="c", subcore_axis_name="s"),
    compiler_params=pltpu.CompilerParams(
        needs_layout_passes=False, use_tc_tiling_on_sc=False),
    scratch_types=[
        pltpu.VMEM((_CHUNK * 3,), jnp.float32),
        pltpu.VMEM((_L, 9 * _CHUNK), jnp.int32),
        pltpu.VMEM((_NSLOT * _CHUNK,), jnp.int32),
        pltpu.VMEM((4, 9 * _CHUNK, 16), jnp.float32),
        pltpu.VMEM((_CHUNK * 2 * _L,), jnp.float32),
        pltpu.SemaphoreType.DMA,
    ],
)(_encode_body)


@jax.jit
def kernel(in_tensor, table):
    tblv = (table.reshape(_L, _T // 128, 128, _D)
            .transpose(0, 1, 3, 2)
            .reshape(_L * _T * _D // 128, 128))
    tbl8 = _interleave(tblv).reshape(_L * _T // 8, 16)
    flat = _encode(in_tensor.reshape(_B * 3), tbl8)
    return flat.reshape(_B, 2 * _L)
